# baseline (speedup 1.0000x reference)
import functools

import jax
import jax.numpy as jnp
from jax.experimental import pallas as pl
from jax.experimental.pallas import tpu as pltpu

_GAMMA = 0.5
_NIN = 3
_NOUT = 2


def _vq_body(x_any, o_any, c_ref, xs, os_, isem, osem, *, h, r, n, d):
    i = pl.program_id(0)
    nsteps = pl.num_programs(0)

    def in_dma(step, buf):
        return pltpu.make_async_copy(
            x_any.at[step // h, step % h], xs.at[buf], isem.at[buf]
        )

    def out_dma(step, buf):
        return pltpu.make_async_copy(
            os_.at[buf], o_any.at[step // h, step % h], osem.at[buf]
        )

    @pl.when(i == 0)
    def _():
        for k in range(_NIN):
            in_dma(k, k).start()

    icur = jax.lax.rem(i, _NIN)
    ocur = jax.lax.rem(i, _NOUT)

    in_dma(i, icur).wait()

    x3 = xs[icur]
    xf = x3.reshape(r * n, d)
    c0 = jnp.sum(x3, axis=1)
    c0 = c0 * jax.lax.rsqrt(jnp.sum(c0 * c0, axis=1, keepdims=True))

    dot = functools.partial(
        jax.lax.dot_general,
        preferred_element_type=jnp.float32,
        precision=jax.lax.Precision.DEFAULT,
    )
    scores0 = dot(xf, c0, dimension_numbers=(((1,), (1,)), ((), ())))
    rowmax = jnp.max(scores0, axis=1, keepdims=True)
    colmax = jnp.max(scores0, axis=0, keepdims=True)
    attn_t = (scores0 == rowmax).astype(jnp.float32) + (
        scores0 == colmax
    ).astype(jnp.float32)
    delta = dot(attn_t, xf, dimension_numbers=(((0,), (0,)), ((), ())))
    delta = delta * jax.lax.rsqrt(jnp.sum(delta * delta, axis=1, keepdims=True))
    c1 = _GAMMA * c0 + (1.0 - _GAMMA) * delta
    c1 = c1 * jax.lax.rsqrt(jnp.sum(c1 * c1, axis=1, keepdims=True))
    c_ref[0, 0] = c1

    scores1 = dot(xf, c1, dimension_numbers=(((1,), (1,)), ((), ())))
    m1 = jnp.max(scores1, axis=1, keepdims=True)
    onehot = (scores1 == m1).astype(jnp.float32)

    @pl.when(i >= _NOUT)
    def _():
        out_dma(i - _NOUT, ocur).wait()

    os_[ocur] = onehot.reshape(r, n, d)
    out_dma(i, ocur).start()

    @pl.when(i + _NIN < nsteps)
    def _():
        in_dma(i + _NIN, icur).start()

    @pl.when(i == nsteps - 1)
    def _():
        out_dma(i, ocur).wait()
        out_dma(i - 1, jax.lax.rem(i - 1, _NOUT)).wait()


def kernel(x):
    b, h, r, n, d = x.shape
    out, c = pl.pallas_call(
        functools.partial(_vq_body, h=h, r=r, n=n, d=d),
        grid=(b * h,),
        in_specs=[pl.BlockSpec(memory_space=pl.ANY)],
        out_specs=[
            pl.BlockSpec(memory_space=pl.ANY),
            pl.BlockSpec((1, 1, r, d), lambda i: (i // h, i % h, 0, 0)),
        ],
        out_shape=[
            jax.ShapeDtypeStruct((b, h, r, n, d), jnp.float32),
            jax.ShapeDtypeStruct((b, h, r, d), jnp.float32),
        ],
        scratch_shapes=[
            pltpu.VMEM((_NIN, r, n, d), jnp.float32),
            pltpu.VMEM((_NOUT, r, n, d), jnp.float32),
            pltpu.SemaphoreType.DMA((_NIN,)),
            pltpu.SemaphoreType.DMA((_NOUT,)),
        ],
    )(x)
    return out, c
